# baseline (device time: 186006 ns/iter reference)
import jax
import jax.numpy as jnp
from jax import lax
from jax.experimental import pallas as pl
from jax.experimental.pallas import tpu as pltpu

N_DEV = 16
SQ = 1024
SKV = 1024
HQ_PER = 8
DH = 128
CHUNK = SQ // N_DEV
SCALE = 0.08838834764831843
NEG = -1e9


def kernel(x, Wq, K_ext, V_ext, Wo):
    my = lax.axis_index("i")
    xb = x[0].astype(jnp.bfloat16)
    Wqb = Wq.astype(jnp.bfloat16)
    Kh = lax.dynamic_slice_in_dim(K_ext[0], my * HQ_PER, HQ_PER, axis=1)
    Vh = lax.dynamic_slice_in_dim(V_ext[0], my * HQ_PER, HQ_PER, axis=1)
    Kh = jnp.transpose(Kh, (1, 0, 2)).astype(jnp.bfloat16)
    Vh = jnp.transpose(Vh, (1, 0, 2)).astype(jnp.bfloat16)
    Wob = Wo.astype(jnp.bfloat16)

    def body(x_ref, wq_ref, k_ref, v_ref, wo_ref, out_ref,
             acc_ref, rs_buf, rs_send_sems, rs_recv_sems,
             ag_send_sems, ag_recv_sems):
        my_pos = lax.axis_index("i")
        left = lax.rem(my_pos - 1 + N_DEV, N_DEV)
        right = lax.rem(my_pos + 1, N_DEV)

        barrier_sem = pltpu.get_barrier_semaphore()
        pl.semaphore_signal(barrier_sem, inc=1, device_id=(left,),
                            device_id_type=pl.DeviceIdType.MESH)
        pl.semaphore_signal(barrier_sem, inc=1, device_id=(right,),
                            device_id_type=pl.DeviceIdType.MESH)
        pl.semaphore_wait(barrier_sem, 2)

        q = jnp.dot(x_ref[:], wq_ref[:],
                    preferred_element_type=jnp.float32).astype(jnp.bfloat16)
        qb4 = (lax.broadcasted_iota(jnp.int32, (SQ, SKV), 0) // 64) % 4
        kb4 = (lax.broadcasted_iota(jnp.int32, (SQ, SKV), 1) // 64) % 4
        mask = qb4 == kb4

        acc_ref[:] = jnp.zeros((SQ, SQ), jnp.float32)
        for h in range(HQ_PER):
            qh = q[:, h * DH:(h + 1) * DH]
            s = lax.dot_general(qh, k_ref[h], (((1,), (1,)), ((), ())),
                                preferred_element_type=jnp.float32) * SCALE
            s = jnp.where(mask, s, NEG)
            m = jnp.max(s, axis=1, keepdims=True)
            p = jnp.exp(s - m)
            w = (p / jnp.sum(p, axis=1, keepdims=True)).astype(jnp.bfloat16)
            ctx = jnp.dot(w, v_ref[h],
                          preferred_element_type=jnp.float32).astype(jnp.bfloat16)
            acc_ref[:] += jnp.dot(ctx, wo_ref[h * DH:(h + 1) * DH, :],
                                  preferred_element_type=jnp.float32)

        for st in range(N_DEV - 1):
            send_idx = lax.rem(my_pos - st + N_DEV, N_DEV)
            rdma = pltpu.make_async_remote_copy(
                src_ref=acc_ref.at[pl.ds(send_idx * CHUNK, CHUNK), :],
                dst_ref=rs_buf.at[st],
                send_sem=rs_send_sems.at[st],
                recv_sem=rs_recv_sems.at[st],
                device_id=(right,),
                device_id_type=pl.DeviceIdType.MESH,
            )
            rdma.start()
            rdma.wait()
            recv_idx = lax.rem(my_pos - st - 1 + N_DEV, N_DEV)
            acc_ref[pl.ds(recv_idx * CHUNK, CHUNK), :] += rs_buf[st]

        own = lax.rem(my_pos + 1, N_DEV)
        out_ref[pl.ds(own * CHUNK, CHUNK), :] = (
            acc_ref[pl.ds(own * CHUNK, CHUNK), :])

        for st in range(N_DEV - 1):
            send_idx = lax.rem(my_pos + 1 - st + N_DEV, N_DEV)
            rdma = pltpu.make_async_remote_copy(
                src_ref=out_ref.at[pl.ds(send_idx * CHUNK, CHUNK), :],
                dst_ref=out_ref.at[pl.ds(send_idx * CHUNK, CHUNK), :],
                send_sem=ag_send_sems.at[st],
                recv_sem=ag_recv_sems.at[st],
                device_id=(right,),
                device_id_type=pl.DeviceIdType.MESH,
            )
            rdma.start()
            rdma.wait()

    out = pl.pallas_call(
        body,
        out_shape=jax.ShapeDtypeStruct((SQ, SQ), jnp.float32),
        in_specs=[pl.BlockSpec(memory_space=pltpu.VMEM)] * 5,
        out_specs=pl.BlockSpec(memory_space=pltpu.VMEM),
        scratch_shapes=[
            pltpu.VMEM((SQ, SQ), jnp.float32),
            pltpu.VMEM((N_DEV - 1, CHUNK, SQ), jnp.float32),
            pltpu.SemaphoreType.DMA((N_DEV - 1,)),
            pltpu.SemaphoreType.DMA((N_DEV - 1,)),
            pltpu.SemaphoreType.DMA((N_DEV - 1,)),
            pltpu.SemaphoreType.DMA((N_DEV - 1,)),
        ],
        compiler_params=pltpu.CompilerParams(collective_id=0),
    )(xb, Wqb, Kh, Vh, Wob)
    return out[None, :, :]


# device time: 106783 ns/iter; 1.7419x vs baseline; 1.7419x over previous
import jax
import jax.numpy as jnp
from jax import lax
from jax.experimental import pallas as pl
from jax.experimental.pallas import tpu as pltpu

N_DEV = 16
SQ = 1024
SKV = 1024
HQ_PER = 8
DH = 128
SCALE = 0.08838834764831843
NEG = -1e9

RS_MASKS = (1, 4, 2, 8)
RS_HALF = (512, 256, 128, 64)
RS_ROFF = (0, 512, 768, 896)
AG_MASKS = (8, 2, 4, 1)
AG_LEN = (64, 128, 256, 512)


def kernel(x, Wq, K_ext, V_ext, Wo):
    my = lax.axis_index("i")
    xb = x[0].astype(jnp.bfloat16)
    Wqb = Wq.astype(jnp.bfloat16)
    Kh = lax.dynamic_slice_in_dim(K_ext[0], my * HQ_PER, HQ_PER, axis=1)
    Vh = lax.dynamic_slice_in_dim(V_ext[0], my * HQ_PER, HQ_PER, axis=1)
    Kh = jnp.transpose(Kh, (1, 0, 2)).astype(jnp.bfloat16)
    Vh = jnp.transpose(Vh, (1, 0, 2)).astype(jnp.bfloat16)
    Wob = Wo.astype(jnp.bfloat16)

    def body(x_ref, wq_ref, k_ref, v_ref, wo_ref, out_ref,
             acc_ref, ag_buf, send_buf, recv_buf,
             rs_send_sems, rs_recv_sems, ag_send_sems, ag_recv_sems):
        my_pos = lax.axis_index("i")

        barrier_sem = pltpu.get_barrier_semaphore()
        for m in (1, 2, 4, 8):
            pl.semaphore_signal(barrier_sem, inc=1,
                                device_id=(jnp.bitwise_xor(my_pos, m),),
                                device_id_type=pl.DeviceIdType.MESH)
        pl.semaphore_wait(barrier_sem, 4)

        q = jnp.dot(x_ref[:], wq_ref[:],
                    preferred_element_type=jnp.float32).astype(jnp.bfloat16)
        qb4 = (lax.broadcasted_iota(jnp.int32, (SQ, SKV), 0) // 64) % 4
        kb4 = (lax.broadcasted_iota(jnp.int32, (SQ, SKV), 1) // 64) % 4
        mask = qb4 == kb4

        acc_ref[:] = jnp.zeros((SQ, SQ), jnp.float32)
        for h in range(HQ_PER):
            qh = q[:, h * DH:(h + 1) * DH]
            s = lax.dot_general(qh, k_ref[h], (((1,), (1,)), ((), ())),
                                preferred_element_type=jnp.float32) * SCALE
            s = jnp.where(mask, s, NEG)
            mx = jnp.max(s, axis=1, keepdims=True)
            p = jnp.exp(s - mx)
            w = (p / jnp.sum(p, axis=1, keepdims=True)).astype(jnp.bfloat16)
            ctx = jnp.dot(w, v_ref[h],
                          preferred_element_type=jnp.float32).astype(jnp.bfloat16)
            acc_ref[:] += jnp.dot(ctx, wo_ref[h * DH:(h + 1) * DH, :],
                                  preferred_element_type=jnp.float32)

        off = jnp.int32(0)
        for k in range(4):
            half = RS_HALF[k]
            partner = jnp.bitwise_xor(my_pos, RS_MASKS[k])
            bit = jnp.bitwise_and(my_pos, RS_MASKS[k]) != 0
            send_off = off + jnp.where(bit, 0, half)
            keep_off = off + jnp.where(bit, half, 0)
            send_buf[pl.ds(0, half), :] = (
                acc_ref[pl.ds(send_off, half), :].astype(jnp.bfloat16))
            rdma = pltpu.make_async_remote_copy(
                src_ref=send_buf.at[pl.ds(0, half), :],
                dst_ref=recv_buf.at[pl.ds(RS_ROFF[k], half), :],
                send_sem=rs_send_sems.at[k],
                recv_sem=rs_recv_sems.at[k],
                device_id=(partner,),
                device_id_type=pl.DeviceIdType.MESH,
            )
            rdma.start()
            rdma.wait()
            acc_ref[pl.ds(keep_off, half), :] += (
                recv_buf[pl.ds(RS_ROFF[k], half), :].astype(jnp.float32))
            off = keep_off

        ag_buf[pl.ds(off, 64), :] = acc_ref[pl.ds(off, 64), :].astype(jnp.bfloat16)

        for k in range(4):
            ln = AG_LEN[k]
            partner = jnp.bitwise_xor(my_pos, AG_MASKS[k])
            bit = jnp.bitwise_and(my_pos, AG_MASKS[k]) != 0
            rdma = pltpu.make_async_remote_copy(
                src_ref=ag_buf.at[pl.ds(off, ln), :],
                dst_ref=ag_buf.at[pl.ds(off, ln), :],
                send_sem=ag_send_sems.at[k],
                recv_sem=ag_recv_sems.at[k],
                device_id=(partner,),
                device_id_type=pl.DeviceIdType.MESH,
            )
            rdma.start()
            rdma.wait()
            off = off - jnp.where(bit, ln, 0)

        out_ref[:] = ag_buf[:].astype(jnp.float32)

    out = pl.pallas_call(
        body,
        out_shape=jax.ShapeDtypeStruct((SQ, SQ), jnp.float32),
        in_specs=[pl.BlockSpec(memory_space=pltpu.VMEM)] * 5,
        out_specs=pl.BlockSpec(memory_space=pltpu.VMEM),
        scratch_shapes=[
            pltpu.VMEM((SQ, SQ), jnp.float32),
            pltpu.VMEM((SQ, SQ), jnp.bfloat16),
            pltpu.VMEM((512, SQ), jnp.bfloat16),
            pltpu.VMEM((960, SQ), jnp.bfloat16),
            pltpu.SemaphoreType.DMA((4,)),
            pltpu.SemaphoreType.DMA((4,)),
            pltpu.SemaphoreType.DMA((4,)),
            pltpu.SemaphoreType.DMA((4,)),
        ],
        compiler_params=pltpu.CompilerParams(collective_id=0),
    )(xb, Wqb, Kh, Vh, Wob)
    return out[None, :, :]
